# baseline (speedup 1.0000x reference)
import functools

import jax
import jax.numpy as jnp
from jax import lax
from jax.experimental import pallas as pl
from jax.experimental.pallas import tpu as pltpu
from jax.experimental.pallas import tpu_sc as plsc

_B = 16384
_D = 32
_NW = 32
_BPW = _B // _NW
_CH = 128
_NCH = _BPW // _CH


def _gather_body(tab_hbm, ids_hbm, out_hbm, idx_v, out_v, sem):
    wid = lax.axis_index("s") * 2 + lax.axis_index("c")
    base = wid * _BPW
    pltpu.sync_copy(ids_hbm.at[wid], idx_v)
    copies = []
    for c in range(_D):
        for ch in range(_NCH):
            copies.append(
                pltpu.async_copy(
                    tab_hbm.at[c].at[idx_v.at[ch]],
                    out_v.at[c, pl.ds(ch * _CH, _CH)],
                    sem,
                )
            )
    for cp in copies:
        cp.wait()
    pltpu.sync_copy(out_v, out_hbm.at[:, pl.ds(base, _BPW)])


@jax.jit
def kernel(user_ids, table):
    ids3 = user_ids.astype(jnp.int32).reshape(_NW, _NCH, _CH)
    mesh = plsc.VectorSubcoreMesh(core_axis_name="c", subcore_axis_name="s")
    k = functools.partial(
        pl.kernel,
        mesh=mesh,
        out_type=jax.ShapeDtypeStruct((_D, _B), jnp.float32),
        scratch_types=[
            pltpu.VMEM((_NCH, _CH), jnp.int32),
            pltpu.VMEM((_D, _BPW), jnp.float32),
            pltpu.SemaphoreType.DMA,
        ],
        compiler_params=pltpu.CompilerParams(use_tc_tiling_on_sc=False),
    )(_gather_body)
    return k(table.T, ids3).T


# speedup vs baseline: 5.8742x; 5.8742x over previous
import functools

import jax
import jax.numpy as jnp
from jax import lax
from jax.experimental import pallas as pl
from jax.experimental.pallas import tpu as pltpu
from jax.experimental.pallas import tpu_sc as plsc

_B = 16384
_D = 32
_V = 1000000
_VP = 1000064
_NW = 32
_BPW = _B // _NW
_CH = 128
_NCH = _BPW // _CH
_TC_FULL = _V // 128
_TPW = 977
_PAIRS = (_TPW + 1) // 2


def _detile_body(tab_hbm, flat_hbm, buf_v, sems):
    wid = lax.axis_index("s") * 2 + lax.axis_index("c")
    c4 = wid // 8
    r = wid % 8
    lo = r * _TPW
    hi = jnp.minimum(lo + _TPW, _TC_FULL)
    row0 = c4 * 8

    def _read(t, slot):
        start = pl.multiple_of(t * 128, 128)
        pltpu.async_copy(
            tab_hbm.at[pl.ds(row0, 8), pl.ds(start, 128)],
            buf_v.at[slot],
            sems.at[slot],
        )

    def _rdwait(slot):
        pltpu.make_async_copy(
            tab_hbm.at[pl.ds(0, 8), pl.ds(0, 128)], buf_v.at[slot], sems.at[slot]
        ).wait()

    def _write(t, slot):
        for c8 in range(8):
            pltpu.async_copy(
                buf_v.at[slot, c8],
                flat_hbm.at[pl.ds((row0 + c8) * _VP + t * 128, 128)],
                sems.at[slot],
            )

    def _wrwait(slot):
        for c8 in range(8):
            pltpu.make_async_copy(
                buf_v.at[slot, c8],
                flat_hbm.at[pl.ds(0, 128)],
                sems.at[slot],
            ).wait()

    _read(lo, 0)

    def _pair(p, _):
        for b in (0, 1):
            t = lo + p * 2 + b

            @pl.when(t < hi)
            def _():
                @pl.when(t + 1 < hi)
                def _():
                    _read(t + 1, 1 - b)

                _rdwait(b)
                _write(t, b)
                _wrwait(b)

        return ()

    lax.fori_loop(0, _PAIRS, _pair, ())

def _gather_body(flat_hbm, ids_hbm, tail_hbm, out_hbm, idx_v, cidx_v, tail_v, out_v, sem):
    wid = lax.axis_index("s") * 2 + lax.axis_index("c")
    base = wid * _BPW
    pltpu.sync_copy(ids_hbm.at[wid], idx_v)
    pltpu.sync_copy(tail_hbm, tail_v.at[:, pl.ds(0, 64)])

    for n in range(_BPW // 16):
        v = idx_v[n // 8, pl.ds((n % 8) * 16, 16)]
        cidx_v[n // 8, pl.ds((n % 8) * 16, 16)] = jnp.minimum(v, _TC_FULL * 128 - 1)

    copies = []
    for c in range(_D):
        for ch in range(_NCH):
            copies.append(
                pltpu.async_copy(
                    flat_hbm.at[pl.ds(c * _VP, _V)].at[cidx_v.at[ch]],
                    out_v.at[c, pl.ds(ch * _CH, _CH)],
                    sem,
                )
            )
    for cp in copies:
        cp.wait()

    iota = lax.iota(jnp.int32, 16)
    for n in range(_BPW // 16):
        v = idx_v[n // 8, pl.ds((n % 8) * 16, 16)]
        sel = v >= _TC_FULL * 128
        toff = jnp.maximum(v - _TC_FULL * 128, 0)
        ivec = iota + n * 16
        for c in range(_D):
            vals = plsc.load_gather(
                tail_v, [jnp.full((16,), 0, jnp.int32) + c, toff], mask=sel
            )
            plsc.store_scatter(out_v, [jnp.full((16,), 0, jnp.int32) + c, ivec],
                               vals, mask=sel)

    pltpu.sync_copy(out_v, out_hbm.at[:, pl.ds(base, _BPW)])


@jax.jit
def kernel(user_ids, table):
    ids3 = user_ids.astype(jnp.int32).reshape(_NW, _NCH, _CH)
    mesh = plsc.VectorSubcoreMesh(core_axis_name="c", subcore_axis_name="s")

    detile = functools.partial(
        pl.kernel,
        mesh=mesh,
        out_type=jax.ShapeDtypeStruct((_D * _VP,), jnp.float32),
        scratch_types=[
            pltpu.VMEM((2, 8, 128), jnp.float32),
            pltpu.SemaphoreType.DMA((2,)),
        ],
    )(_detile_body)
    flat = detile(table.T)
    tailT = table.T[:, _TC_FULL * 128:]

    gather = functools.partial(
        pl.kernel,
        mesh=mesh,
        out_type=jax.ShapeDtypeStruct((_D, _B), jnp.float32),
        scratch_types=[
            pltpu.VMEM((_NCH, _CH), jnp.int32),
            pltpu.VMEM((_NCH, _CH), jnp.int32),
            pltpu.VMEM((_D, 128), jnp.float32),
            pltpu.VMEM((_D, _BPW), jnp.float32),
            pltpu.SemaphoreType.DMA,
        ],
        compiler_params=pltpu.CompilerParams(use_tc_tiling_on_sc=False, needs_layout_passes=False),
    )(_gather_body)
    return gather(flat, ids3, tailT).T


# speedup vs baseline: 16.5618x; 2.8194x over previous
import functools

import jax
import jax.numpy as jnp
from jax import lax
from jax.experimental import pallas as pl
from jax.experimental.pallas import tpu as pltpu
from jax.experimental.pallas import tpu_sc as plsc

_B = 16384
_D = 32
_V = 1000000
_VP = 1000064
_NW = 32
_BPW = _B // _NW
_CH = 128
_NCH = _BPW // _CH
_TC_FULL = _V // 128
_TPW = 977
_K = 16


def _detile_body(tab_hbm, flat_hbm, buf_v, sems):
    wid = lax.axis_index("s") * 2 + lax.axis_index("c")
    c4 = wid // 8
    r = wid % 8
    lo = r * _TPW
    hi = jnp.minimum(lo + _TPW, _TC_FULL)
    row0 = c4 * 8
    nsteps = (_TPW + _K - 1) // _K

    def _start(s):
        return pl.multiple_of(jnp.maximum(jnp.minimum(lo + s * _K, hi - _K), 0) * 128, 128)

    def _read(s, slot):
        pltpu.async_copy(
            tab_hbm.at[pl.ds(row0, 8), pl.ds(_start(s), _K * 128)],
            buf_v.at[slot],
            sems.at[slot],
        )

    def _rdwait(slot):
        pltpu.make_async_copy(
            tab_hbm.at[pl.ds(0, 8), pl.ds(0, _K * 128)], buf_v.at[slot], sems.at[slot]
        ).wait()

    def _write(s, slot):
        for c8 in range(8):
            pltpu.async_copy(
                buf_v.at[slot, c8],
                flat_hbm.at[pl.ds((row0 + c8) * _VP + _start(s), _K * 128)],
                sems.at[slot],
            )

    def _wrwait(slot):
        for c8 in range(8):
            pltpu.make_async_copy(
                buf_v.at[slot, c8],
                flat_hbm.at[pl.ds(0, _K * 128)],
                sems.at[slot],
            ).wait()

    _read(0, 0)

    def _pair(p, _):
        for b in (0, 1):
            s = p * 2 + b

            @pl.when(s < nsteps)
            def _():
                @pl.when(s + 1 < nsteps)
                def _():
                    _read(s + 1, 1 - b)

                _rdwait(b)
                _write(s, b)
                _wrwait(b)

        return ()

    lax.fori_loop(0, (nsteps + 1) // 2, _pair, ())

def _gather_body(flat_hbm, ids_hbm, tail_hbm, out_hbm, idx_v, cidx_v, tail_v, out_v, sem):
    wid = lax.axis_index("s") * 2 + lax.axis_index("c")
    base = wid * _BPW
    pltpu.sync_copy(ids_hbm.at[wid], idx_v)
    pltpu.sync_copy(tail_hbm, tail_v.at[:, pl.ds(0, 64)])

    for n in range(_BPW // 16):
        v = idx_v[n // 8, pl.ds((n % 8) * 16, 16)]
        cidx_v[n // 8, pl.ds((n % 8) * 16, 16)] = jnp.minimum(v, _TC_FULL * 128 - 1)

    copies = []
    for c in range(_D):
        for ch in range(_NCH):
            copies.append(
                pltpu.async_copy(
                    flat_hbm.at[pl.ds(c * _VP, _V)].at[cidx_v.at[ch]],
                    out_v.at[c, pl.ds(ch * _CH, _CH)],
                    sem,
                )
            )
    for cp in copies:
        cp.wait()

    iota = lax.iota(jnp.int32, 16)
    for n in range(_BPW // 16):
        v = idx_v[n // 8, pl.ds((n % 8) * 16, 16)]
        sel = v >= _TC_FULL * 128
        toff = jnp.maximum(v - _TC_FULL * 128, 0)
        ivec = iota + n * 16
        for c in range(_D):
            vals = plsc.load_gather(
                tail_v, [jnp.full((16,), 0, jnp.int32) + c, toff], mask=sel
            )
            plsc.store_scatter(out_v, [jnp.full((16,), 0, jnp.int32) + c, ivec],
                               vals, mask=sel)

    pltpu.sync_copy(out_v, out_hbm.at[:, pl.ds(base, _BPW)])


@jax.jit
def kernel(user_ids, table):
    ids3 = user_ids.astype(jnp.int32).reshape(_NW, _NCH, _CH)
    mesh = plsc.VectorSubcoreMesh(core_axis_name="c", subcore_axis_name="s")

    detile = functools.partial(
        pl.kernel,
        mesh=mesh,
        out_type=jax.ShapeDtypeStruct((_D * _VP,), jnp.float32),
        scratch_types=[
            pltpu.VMEM((2, 8, _K * 128), jnp.float32),
            pltpu.SemaphoreType.DMA((2,)),
        ],
    )(_detile_body)
    flat = detile(table.T)
    tailT = table.T[:, _TC_FULL * 128:]

    gather = functools.partial(
        pl.kernel,
        mesh=mesh,
        out_type=jax.ShapeDtypeStruct((_D, _B), jnp.float32),
        scratch_types=[
            pltpu.VMEM((_NCH, _CH), jnp.int32),
            pltpu.VMEM((_NCH, _CH), jnp.int32),
            pltpu.VMEM((_D, 128), jnp.float32),
            pltpu.VMEM((_D, _BPW), jnp.float32),
            pltpu.SemaphoreType.DMA,
        ],
        compiler_params=pltpu.CompilerParams(use_tc_tiling_on_sc=False, needs_layout_passes=False),
    )(_gather_body)
    return gather(flat, ids3, tailT).T


# speedup vs baseline: 16.6889x; 1.0077x over previous
import functools

import jax
import jax.numpy as jnp
from jax import lax
from jax.experimental import pallas as pl
from jax.experimental.pallas import tpu as pltpu
from jax.experimental.pallas import tpu_sc as plsc

_B = 16384
_D = 32
_V = 1000000
_VP = 1000064
_NW = 32
_BPW = _B // _NW
_CH = 128
_NCH = _BPW // _CH
_TC_FULL = _V // 128
_TPW = 977
_K = 32


def _detile_body(tab_hbm, flat_hbm, buf_v, sems):
    wid = lax.axis_index("s") * 2 + lax.axis_index("c")
    c4 = wid // 8
    r = wid % 8
    lo = r * _TPW
    hi = jnp.minimum(lo + _TPW, _TC_FULL)
    row0 = c4 * 8
    nsteps = (_TPW + _K - 1) // _K

    def _start(s):
        return pl.multiple_of(jnp.maximum(jnp.minimum(lo + s * _K, hi - _K), 0) * 128, 128)

    def _read(s, slot):
        pltpu.async_copy(
            tab_hbm.at[pl.ds(row0, 8), pl.ds(_start(s), _K * 128)],
            buf_v.at[slot],
            sems.at[slot],
        )

    def _rdwait(slot):
        pltpu.make_async_copy(
            tab_hbm.at[pl.ds(0, 8), pl.ds(0, _K * 128)], buf_v.at[slot], sems.at[slot]
        ).wait()

    def _write(s, slot):
        for c8 in range(8):
            pltpu.async_copy(
                buf_v.at[slot, c8],
                flat_hbm.at[pl.ds((row0 + c8) * _VP + _start(s), _K * 128)],
                sems.at[slot],
            )

    def _wrwait(slot):
        for c8 in range(8):
            pltpu.make_async_copy(
                buf_v.at[slot, c8],
                flat_hbm.at[pl.ds(0, _K * 128)],
                sems.at[slot],
            ).wait()

    _read(0, 0)

    def _pair(p, _):
        for b in (0, 1):
            s = p * 2 + b

            @pl.when(s < nsteps)
            def _():
                @pl.when(s + 1 < nsteps)
                def _():
                    _read(s + 1, 1 - b)

                _rdwait(b)
                _write(s, b)
                _wrwait(b)

        return ()

    lax.fori_loop(0, (nsteps + 1) // 2, _pair, ())

def _gather_body(flat_hbm, ids_hbm, tail_hbm, out_hbm, idx_v, cidx_v, tail_v, out_v, sem):
    wid = lax.axis_index("s") * 2 + lax.axis_index("c")
    base = wid * _BPW
    pltpu.sync_copy(ids_hbm.at[wid], idx_v)
    pltpu.sync_copy(tail_hbm, tail_v.at[:, pl.ds(0, 64)])

    for n in range(_BPW // 16):
        v = idx_v[n // 8, pl.ds((n % 8) * 16, 16)]
        cidx_v[n // 8, pl.ds((n % 8) * 16, 16)] = jnp.minimum(v, _TC_FULL * 128 - 1)

    copies = []
    for c in range(_D):
        for ch in range(_NCH):
            copies.append(
                pltpu.async_copy(
                    flat_hbm.at[pl.ds(c * _VP, _V)].at[cidx_v.at[ch]],
                    out_v.at[c, pl.ds(ch * _CH, _CH)],
                    sem,
                )
            )
    for cp in copies:
        cp.wait()

    iota = lax.iota(jnp.int32, 16)
    for n in range(_BPW // 16):
        v = idx_v[n // 8, pl.ds((n % 8) * 16, 16)]
        sel = v >= _TC_FULL * 128
        toff = jnp.maximum(v - _TC_FULL * 128, 0)
        ivec = iota + n * 16
        for c in range(_D):
            vals = plsc.load_gather(
                tail_v, [jnp.full((16,), 0, jnp.int32) + c, toff], mask=sel
            )
            plsc.store_scatter(out_v, [jnp.full((16,), 0, jnp.int32) + c, ivec],
                               vals, mask=sel)

    pltpu.sync_copy(out_v, out_hbm.at[:, pl.ds(base, _BPW)])


@jax.jit
def kernel(user_ids, table):
    ids3 = user_ids.astype(jnp.int32).reshape(_NW, _NCH, _CH)
    mesh = plsc.VectorSubcoreMesh(core_axis_name="c", subcore_axis_name="s")

    detile = functools.partial(
        pl.kernel,
        mesh=mesh,
        out_type=jax.ShapeDtypeStruct((_D * _VP,), jnp.float32),
        scratch_types=[
            pltpu.VMEM((2, 8, _K * 128), jnp.float32),
            pltpu.SemaphoreType.DMA((2,)),
        ],
    )(_detile_body)
    flat = detile(table.T)
    tailT = table.T[:, _TC_FULL * 128:]

    gather = functools.partial(
        pl.kernel,
        mesh=mesh,
        out_type=jax.ShapeDtypeStruct((_D, _B), jnp.float32),
        scratch_types=[
            pltpu.VMEM((_NCH, _CH), jnp.int32),
            pltpu.VMEM((_NCH, _CH), jnp.int32),
            pltpu.VMEM((_D, 128), jnp.float32),
            pltpu.VMEM((_D, _BPW), jnp.float32),
            pltpu.SemaphoreType.DMA,
        ],
        compiler_params=pltpu.CompilerParams(use_tc_tiling_on_sc=False, needs_layout_passes=False),
    )(_gather_body)
    return gather(flat, ids3, tailT).T


# speedup vs baseline: 16.7623x; 1.0044x over previous
import functools

import jax
import jax.numpy as jnp
from jax import lax
from jax.experimental import pallas as pl
from jax.experimental.pallas import tpu as pltpu
from jax.experimental.pallas import tpu_sc as plsc

_B = 16384
_D = 32
_V = 1000000
_VP = 1000064
_NW = 32
_BPW = _B // _NW
_CH = 128
_NCH = _BPW // _CH
_TC_FULL = _V // 128
_TPW = 977
_K = 48


def _detile_body(tab_hbm, flat_hbm, buf_v, sems):
    wid = lax.axis_index("s") * 2 + lax.axis_index("c")
    c4 = wid // 8
    r = wid % 8
    lo = r * _TPW
    hi = jnp.minimum(lo + _TPW, _TC_FULL)
    row0 = c4 * 8
    nsteps = (_TPW + _K - 1) // _K

    def _start(s):
        return pl.multiple_of(jnp.maximum(jnp.minimum(lo + s * _K, hi - _K), 0) * 128, 128)

    def _read(s, slot):
        pltpu.async_copy(
            tab_hbm.at[pl.ds(row0, 8), pl.ds(_start(s), _K * 128)],
            buf_v.at[slot],
            sems.at[slot],
        )

    def _rdwait(slot):
        pltpu.make_async_copy(
            tab_hbm.at[pl.ds(0, 8), pl.ds(0, _K * 128)], buf_v.at[slot], sems.at[slot]
        ).wait()

    def _write(s, slot):
        for c8 in range(8):
            pltpu.async_copy(
                buf_v.at[slot, c8],
                flat_hbm.at[pl.ds((row0 + c8) * _VP + _start(s), _K * 128)],
                sems.at[slot],
            )

    def _wrwait(slot):
        for c8 in range(8):
            pltpu.make_async_copy(
                buf_v.at[slot, c8],
                flat_hbm.at[pl.ds(0, _K * 128)],
                sems.at[slot],
            ).wait()

    _read(0, 0)

    def _pair(p, _):
        for b in (0, 1):
            s = p * 2 + b

            @pl.when(s < nsteps)
            def _():
                @pl.when(s + 1 < nsteps)
                def _():
                    _read(s + 1, 1 - b)

                _rdwait(b)
                _write(s, b)
                _wrwait(b)

        return ()

    lax.fori_loop(0, (nsteps + 1) // 2, _pair, ())

def _gather_body(flat_hbm, ids_hbm, tail_hbm, out_hbm, idx_v, cidx_v, tail_v, out_v, sem):
    wid = lax.axis_index("s") * 2 + lax.axis_index("c")
    base = wid * _BPW
    pltpu.sync_copy(ids_hbm.at[wid], idx_v)
    pltpu.sync_copy(tail_hbm, tail_v.at[:, pl.ds(0, 64)])

    for n in range(_BPW // 16):
        v = idx_v[n // 8, pl.ds((n % 8) * 16, 16)]
        cidx_v[n // 8, pl.ds((n % 8) * 16, 16)] = jnp.minimum(v, _TC_FULL * 128 - 1)

    copies = []
    for c in range(_D):
        for ch in range(_NCH):
            copies.append(
                pltpu.async_copy(
                    flat_hbm.at[pl.ds(c * _VP, _V)].at[cidx_v.at[ch]],
                    out_v.at[c, pl.ds(ch * _CH, _CH)],
                    sem,
                )
            )
    for cp in copies:
        cp.wait()

    iota = lax.iota(jnp.int32, 16)
    for n in range(_BPW // 16):
        v = idx_v[n // 8, pl.ds((n % 8) * 16, 16)]
        sel = v >= _TC_FULL * 128
        toff = jnp.maximum(v - _TC_FULL * 128, 0)
        ivec = iota + n * 16
        for c in range(_D):
            vals = plsc.load_gather(
                tail_v, [jnp.full((16,), 0, jnp.int32) + c, toff], mask=sel
            )
            plsc.store_scatter(out_v, [jnp.full((16,), 0, jnp.int32) + c, ivec],
                               vals, mask=sel)

    pltpu.sync_copy(out_v, out_hbm.at[:, pl.ds(base, _BPW)])


@jax.jit
def kernel(user_ids, table):
    ids3 = user_ids.astype(jnp.int32).reshape(_NW, _NCH, _CH)
    mesh = plsc.VectorSubcoreMesh(core_axis_name="c", subcore_axis_name="s")

    detile = functools.partial(
        pl.kernel,
        mesh=mesh,
        out_type=jax.ShapeDtypeStruct((_D * _VP,), jnp.float32),
        scratch_types=[
            pltpu.VMEM((2, 8, _K * 128), jnp.float32),
            pltpu.SemaphoreType.DMA((2,)),
        ],
    )(_detile_body)
    flat = detile(table.T)
    tailT = table.T[:, _TC_FULL * 128:]

    gather = functools.partial(
        pl.kernel,
        mesh=mesh,
        out_type=jax.ShapeDtypeStruct((_D, _B), jnp.float32),
        scratch_types=[
            pltpu.VMEM((_NCH, _CH), jnp.int32),
            pltpu.VMEM((_NCH, _CH), jnp.int32),
            pltpu.VMEM((_D, 128), jnp.float32),
            pltpu.VMEM((_D, _BPW), jnp.float32),
            pltpu.SemaphoreType.DMA,
        ],
        compiler_params=pltpu.CompilerParams(use_tc_tiling_on_sc=False, needs_layout_passes=False),
    )(_gather_body)
    return gather(flat, ids3, tailT).T
